# baseline (speedup 1.0000x reference)
import functools

import jax
import jax.numpy as jnp
from jax import lax
from jax.experimental import pallas as pl
from jax.experimental.pallas import tpu as pltpu
from jax.experimental.pallas import tpu_sc as plsc

_NC, _NS = 2, 16
_NW = _NC * _NS
_SUB = 125


_GSUB = 100
_GF = 2
_GBIG = _GSUB * _GF


@functools.lru_cache(maxsize=None)
def _make_gather(n_edges, n_nodes, d):
    epw = n_edges // _NW
    nbig = epw // _GBIG
    mesh = plsc.VectorSubcoreMesh(
        core_axis_name="c", subcore_axis_name="s",
        num_cores=_NC, num_subcores=_NS)

    @functools.partial(
        pl.kernel,
        out_type=jax.ShapeDtypeStruct((n_edges, d), jnp.float32),
        mesh=mesh,
        scratch_types=[
            pltpu.VMEM((2, _GF, _GSUB), jnp.int32),
            pltpu.VMEM((2, _GF, _GSUB), jnp.int32),
            pltpu.VMEM((2, _GBIG, d), jnp.float32),
            pltpu.SemaphoreType.DMA,
            pltpu.SemaphoreType.DMA,
            pltpu.SemaphoreType.DMA,
            pltpu.SemaphoreType.DMA,
        ],
        compiler_params=pltpu.CompilerParams(use_tc_tiling_on_sc=False),
    )
    def gather_k(pa_hbm, pb_hbm, src_hbm, dst_hbm, g_hbm,
                 idx_a, idx_b, rows, isem, asem, bsem, wsem):
        wid = lax.axis_index("s") * _NC + lax.axis_index("c")
        base = pl.multiple_of(wid * epw, 8)

        def fire_idx(k):
            b = k % 2
            return [pltpu.async_copy(src_hbm.at[wid, k], idx_a.at[b], isem),
                    pltpu.async_copy(dst_hbm.at[wid, k], idx_b.at[b], isem)]

        wdesc = [None] * nbig
        idesc = fire_idx(0)
        for k in range(nbig):
            b = k % 2
            if k >= 2:
                wdesc[k - 2].wait()
            for dsc in idesc:
                dsc.wait()
            pa_descs = [
                pltpu.async_copy(
                    pa_hbm.at[idx_a.at[b, j]],
                    rows.at[b, pl.ds(j * _GSUB, _GSUB)], asem)
                for j in range(_GF)
            ]
            if k + 1 < nbig:
                idesc = fire_idx(k + 1)
            pb_descs = []
            for j in range(_GF):
                pa_descs[j].wait()
                pb_descs.append(pltpu.async_copy(
                    pb_hbm.at[idx_b.at[b, j]],
                    rows.at[b, pl.ds(j * _GSUB, _GSUB)], bsem, add=True))
            for dsc in pb_descs:
                dsc.wait()
            off = pl.multiple_of(base + k * _GBIG, 8)
            wdesc[k] = pltpu.async_copy(
                rows.at[b], g_hbm.at[pl.ds(off, _GBIG)], wsem)
        wdesc[nbig - 2].wait()
        wdesc[nbig - 1].wait()

    return gather_k


@functools.lru_cache(maxsize=None)
def _make_scatter(n_edges, n_nodes, de):
    eph = n_edges // _NC
    ch = 8000
    nch = eph // ch
    mesh = plsc.VectorSubcoreMesh(
        core_axis_name="c", subcore_axis_name="s",
        num_cores=_NC, num_subcores=_NS)

    @functools.partial(
        pl.kernel,
        out_type=jax.ShapeDtypeStruct((_NC, de, n_nodes), jnp.float32),
        mesh=mesh,
        scratch_types=[
            pltpu.VMEM((2, ch), jnp.int32),
            pltpu.VMEM((2, ch), jnp.float32),
            pltpu.VMEM((n_nodes,), jnp.float32),
            pltpu.SemaphoreType.DMA,
            pltpu.SemaphoreType.DMA,
        ],
        compiler_params=pltpu.CompilerParams(use_tc_tiling_on_sc=False,
                                             needs_layout_passes=False),
    )
    def scatter_k(et_hbm, dst_hbm, zeros_hbm, out_hbm, idx, vals, acc,
                  isem, vsem):
        cid = lax.axis_index("c")
        sid = lax.axis_index("s")
        base = cid * eph
        pltpu.sync_copy(zeros_hbm, acc)

        def fire(k):
            b = k % 2
            off = pl.multiple_of(base + k * ch, 8)
            return [
                pltpu.async_copy(dst_hbm.at[pl.ds(off, ch)], idx.at[b], isem),
                pltpu.async_copy(et_hbm.at[sid, pl.ds(off, ch)], vals.at[b],
                                 vsem),
            ]

        descs = fire(0)
        for k in range(nch):
            b = k % 2
            for dsc in descs:
                dsc.wait()
            if k + 1 < nch:
                descs = fire(k + 1)

            def body(i, carry):
                iv = idx[b, pl.ds(i * 16, 16)]
                vv = vals[b, pl.ds(i * 16, 16)]
                plsc.addupdate_scatter(acc, [iv], vv)
                return carry

            lax.fori_loop(0, ch // 16, body, 0, unroll=16)

        pltpu.sync_copy(acc, out_hbm.at[cid, sid])

    return scatter_k


_BF = jnp.bfloat16


def _dot(a, b):
    return jnp.dot(a.astype(_BF), b.astype(_BF),
                   preferred_element_type=jnp.float32)


def _edge_body(g_ref, eat_ref, w1c, b1, w2, b2, w3, b3t, lst, lbt, out_ref):
    g = g_ref[...]
    eat = eat_ref[...]
    t1 = lax.dot_general(eat.astype(_BF), w1c[...].astype(_BF),
                         (((0,), (0,)), ((), ())),
                         preferred_element_type=jnp.float32)
    h = g + t1 + b1[...]
    h = jnp.maximum(h, 0.0)
    h = _dot(h, w2[...]) + b2[...]
    h = jnp.maximum(h, 0.0)
    h3t = lax.dot_general(w3[...].astype(_BF), h.astype(_BF),
                          (((0,), (1,)), ((), ())),
                          preferred_element_type=jnp.float32)
    h3t = h3t + b3t[...]
    mu = jnp.mean(h3t, axis=0, keepdims=True)
    hc = h3t - mu
    var = jnp.mean(hc * hc, axis=0, keepdims=True)
    out_ref[...] = eat + hc * lax.rsqrt(var + 1e-5) * lst[...] + lbt[...]


@functools.lru_cache(maxsize=None)
def _make_edge_mlp(n_edges, d, de, h_dim, block_e):
    grid = (n_edges // block_e,)
    full = lambda shape: pl.BlockSpec(shape, lambda i: (0,) * len(shape))
    return pl.pallas_call(
        _edge_body,
        grid=grid,
        in_specs=[
            pl.BlockSpec((block_e, d), lambda i: (i, 0)),
            pl.BlockSpec((de, block_e), lambda i: (0, i)),
            full((de, h_dim)), full((1, h_dim)),
            full((h_dim, h_dim)), full((1, h_dim)),
            full((h_dim, de)), full((de, 1)),
            full((de, 1)), full((de, 1)),
        ],
        out_specs=pl.BlockSpec((de, block_e), lambda i: (0, i)),
        out_shape=jax.ShapeDtypeStruct((de, n_edges), jnp.float32),
    )


def _agg_term(agg_ref, w1a, i, block_n):
    del i, block_n
    aggt = agg_ref[0] + agg_ref[1]
    return lax.dot_general(aggt.astype(_BF), w1a[...].astype(_BF),
                           (((0,), (0,)), ((), ())),
                           preferred_element_type=jnp.float32)


def _node_body_proj(x_ref, agg_ref, w1x, w1a, b1, w2, b2, w3, b3, ls, lb,
                    wa, wb, out_ref, pa_ref, pb_ref):
    x = x_ref[...]
    h = (_dot(x, w1x[...])
         + _agg_term(agg_ref, w1a, pl.program_id(0), x.shape[0]) + b1[...])
    h = jnp.maximum(h, 0.0)
    h = _dot(h, w2[...]) + b2[...]
    h = jnp.maximum(h, 0.0)
    h = _dot(h, w3[...]) + b3[...]
    mu = jnp.mean(h, axis=-1, keepdims=True)
    hc = h - mu
    var = jnp.mean(hc * hc, axis=-1, keepdims=True)
    xn = x + hc * lax.rsqrt(var + 1e-5) * ls[...] + lb[...]
    out_ref[...] = xn
    pa_ref[...] = _dot(xn, wa[...])
    pb_ref[...] = _dot(xn, wb[...])


def _node_body_last(x_ref, agg_ref, w1x, w1a, b1, w2, b2, w3, b3, ls, lb,
                    out_ref):
    x = x_ref[...]
    h = (_dot(x, w1x[...])
         + _agg_term(agg_ref, w1a, pl.program_id(0), x.shape[0]) + b1[...])
    h = jnp.maximum(h, 0.0)
    h = _dot(h, w2[...]) + b2[...]
    h = jnp.maximum(h, 0.0)
    h = _dot(h, w3[...]) + b3[...]
    mu = jnp.mean(h, axis=-1, keepdims=True)
    hc = h - mu
    var = jnp.mean(hc * hc, axis=-1, keepdims=True)
    out_ref[...] = x + hc * lax.rsqrt(var + 1e-5) * ls[...] + lb[...]


@functools.lru_cache(maxsize=None)
def _make_node_mlp(n_nodes, d, de, h_dim, block_n, with_proj):
    grid = (n_nodes // block_n,)
    full = lambda shape: pl.BlockSpec(shape, lambda i: (0,) * len(shape))
    in_specs = [
        pl.BlockSpec((block_n, d), lambda i: (i, 0)),
        full((_NC, de, n_nodes)),
        full((d, h_dim)), full((de, h_dim)), full((1, h_dim)),
        full((h_dim, h_dim)), full((1, h_dim)),
        full((h_dim, d)), full((1, d)),
        full((1, d)), full((1, d)),
    ]
    if with_proj:
        in_specs += [full((d, h_dim)), full((d, h_dim))]
        return pl.pallas_call(
            _node_body_proj,
            grid=grid,
            in_specs=in_specs,
            out_specs=[pl.BlockSpec((block_n, d), lambda i: (i, 0))] * 3,
            out_shape=[jax.ShapeDtypeStruct((n_nodes, d), jnp.float32)] * 3,
        )
    return pl.pallas_call(
        _node_body_last,
        grid=grid,
        in_specs=in_specs,
        out_specs=pl.BlockSpec((block_n, d), lambda i: (i, 0)),
        out_shape=jax.ShapeDtypeStruct((n_nodes, d), jnp.float32),
    )


def _proj_body(x_ref, wa, wb, pa_ref, pb_ref):
    x = x_ref[...]
    pa_ref[...] = _dot(x, wa[...])
    pb_ref[...] = _dot(x, wb[...])


@functools.lru_cache(maxsize=None)
def _make_proj(n_nodes, d, h_dim, block_n):
    grid = (n_nodes // block_n,)
    full = lambda shape: pl.BlockSpec(shape, lambda i: (0,) * len(shape))
    return pl.pallas_call(
        _proj_body,
        grid=grid,
        in_specs=[
            pl.BlockSpec((block_n, d), lambda i: (i, 0)),
            full((d, h_dim)), full((d, h_dim)),
        ],
        out_specs=[pl.BlockSpec((block_n, d), lambda i: (i, 0))] * 2,
        out_shape=[jax.ShapeDtypeStruct((n_nodes, d), jnp.float32)] * 2,
    )


def kernel(x, edge_index, edge_attr,
           eW1, eb1, eW2, eb2, eW3, eb3, eLs, eLb,
           nW1, nb1, nW2, nb2, nW3, nb3, nLs, nLb):
    n_nodes, d = x.shape
    n_edges, de = edge_attr.shape
    nb, _, h_dim = eW2.shape
    epw = n_edges // _NW
    nsub = epw // _SUB
    nbig = epw // _GBIG

    src = edge_index[0].reshape(_NW, nbig, _GF, _GSUB)
    dst = edge_index[1].reshape(_NW, nbig, _GF, _GSUB)
    dstf = edge_index[1]
    zeros = jnp.zeros((n_nodes,), jnp.float32)

    eW1a = eW1[:, :d, :]
    eW1b = eW1[:, d:2 * d, :]
    eW1c = eW1[:, 2 * d:, :]
    nW1x = nW1[:, :d, :]
    nW1a = nW1[:, d:, :]
    r1 = lambda a: a.reshape(a.shape[0], 1, a.shape[-1])
    rt = lambda a: a.reshape(a.shape[0], a.shape[-1], 1)
    eb1r, eb2r = map(r1, (eb1, eb2))
    eb3r, eLsr, eLbr = map(rt, (eb3, eLs, eLb))
    nb1r, nb2r, nb3r, nLsr, nLbr = map(r1, (nb1, nb2, nb3, nLs, nLb))

    gather = _make_gather(n_edges, n_nodes, d)
    scatter = _make_scatter(n_edges, n_nodes, de)
    edge_mlp = _make_edge_mlp(n_edges, d, de, h_dim, 6400)
    node_mlp = _make_node_mlp(n_nodes, d, de, h_dim, n_nodes, True)
    node_mlp_last = _make_node_mlp(n_nodes, d, de, h_dim, n_nodes, False)
    proj = _make_proj(n_nodes, d, h_dim, 2000)

    pa, pb = proj(x, eW1a[0], eW1b[0])
    ea = edge_attr.T
    for i in range(nb):
        g = gather(pa, pb, src, dst)
        ea = edge_mlp(g, ea, eW1c[i], eb1r[i], eW2[i], eb2r[i],
                      eW3[i], eb3r[i], eLsr[i], eLbr[i])
        aggp = scatter(ea, dstf, zeros)
        if i + 1 < nb:
            x, pa, pb = node_mlp(x, aggp, nW1x[i], nW1a[i], nb1r[i],
                                 nW2[i], nb2r[i], nW3[i], nb3r[i],
                                 nLsr[i], nLbr[i], eW1a[i + 1], eW1b[i + 1])
        else:
            x = node_mlp_last(x, aggp, nW1x[i], nW1a[i], nb1r[i],
                              nW2[i], nb2r[i], nW3[i], nb3r[i],
                              nLsr[i], nLbr[i])
    return x


# speedup vs baseline: 1.0260x; 1.0260x over previous
import functools

import jax
import jax.numpy as jnp
from jax import lax
from jax.experimental import pallas as pl
from jax.experimental.pallas import tpu as pltpu
from jax.experimental.pallas import tpu_sc as plsc

_NC, _NS = 2, 16
_NW = _NC * _NS
_SUB = 125


_GSUB = 100
_GF = 2
_GBIG = _GSUB * _GF


@functools.lru_cache(maxsize=None)
def _make_gather(n_edges, n_nodes, d):
    epw = n_edges // _NW
    nbig = epw // _GBIG
    mesh = plsc.VectorSubcoreMesh(
        core_axis_name="c", subcore_axis_name="s",
        num_cores=_NC, num_subcores=_NS)

    @functools.partial(
        pl.kernel,
        out_type=jax.ShapeDtypeStruct((n_edges, d), jnp.float32),
        mesh=mesh,
        scratch_types=[
            pltpu.VMEM((2, _GF, _GSUB), jnp.int32),
            pltpu.VMEM((2, _GF, _GSUB), jnp.int32),
            pltpu.VMEM((2, _GBIG, d), jnp.float32),
            pltpu.SemaphoreType.DMA,
            pltpu.SemaphoreType.DMA,
            pltpu.SemaphoreType.DMA,
            pltpu.SemaphoreType.DMA,
        ],
        compiler_params=pltpu.CompilerParams(use_tc_tiling_on_sc=False),
    )
    def gather_k(pa_hbm, pb_hbm, src_hbm, dst_hbm, g_hbm,
                 idx_a, idx_b, rows, isem, asem, bsem, wsem):
        wid = lax.axis_index("s") * _NC + lax.axis_index("c")
        base = pl.multiple_of(wid * epw, 8)

        def fire_idx(k):
            b = k % 2
            return [pltpu.async_copy(src_hbm.at[wid, k], idx_a.at[b], isem),
                    pltpu.async_copy(dst_hbm.at[wid, k], idx_b.at[b], isem)]

        wdesc = [None] * nbig
        idesc = fire_idx(0)
        for k in range(nbig):
            b = k % 2
            if k >= 2:
                wdesc[k - 2].wait()
            for dsc in idesc:
                dsc.wait()
            pa_descs = [
                pltpu.async_copy(
                    pa_hbm.at[idx_a.at[b, j]],
                    rows.at[b, pl.ds(j * _GSUB, _GSUB)], asem)
                for j in range(_GF)
            ]
            if k + 1 < nbig:
                idesc = fire_idx(k + 1)
            pb_descs = []
            for j in range(_GF):
                pa_descs[j].wait()
                pb_descs.append(pltpu.async_copy(
                    pb_hbm.at[idx_b.at[b, j]],
                    rows.at[b, pl.ds(j * _GSUB, _GSUB)], bsem, add=True))
            for dsc in pb_descs:
                dsc.wait()
            off = pl.multiple_of(base + k * _GBIG, 8)
            wdesc[k] = pltpu.async_copy(
                rows.at[b], g_hbm.at[pl.ds(off, _GBIG)], wsem)
        wdesc[nbig - 2].wait()
        wdesc[nbig - 1].wait()

    return gather_k


@functools.lru_cache(maxsize=None)
def _make_scatter(n_edges, n_nodes, de):
    eph = n_edges // _NC
    nch = -(-eph // 8000)
    while eph % nch or (eph // nch) % 16:
        nch += 1
    ch = eph // nch
    mesh = plsc.VectorSubcoreMesh(
        core_axis_name="c", subcore_axis_name="s",
        num_cores=_NC, num_subcores=_NS)

    @functools.partial(
        pl.kernel,
        out_type=jax.ShapeDtypeStruct((_NC, de, n_nodes), jnp.float32),
        mesh=mesh,
        scratch_types=[
            pltpu.VMEM((2, ch), jnp.int32),
            pltpu.VMEM((2, ch), jnp.float32),
            pltpu.VMEM((n_nodes,), jnp.float32),
            pltpu.SemaphoreType.DMA,
            pltpu.SemaphoreType.DMA,
        ],
        compiler_params=pltpu.CompilerParams(use_tc_tiling_on_sc=False,
                                             needs_layout_passes=False),
    )
    def scatter_k(et_hbm, dst_hbm, zeros_hbm, out_hbm, idx, vals, acc,
                  isem, vsem):
        cid = lax.axis_index("c")
        sid = lax.axis_index("s")
        base = cid * eph
        pltpu.sync_copy(zeros_hbm, acc)

        def fire(k):
            b = k % 2
            off = pl.multiple_of(base + k * ch, 8)
            return [
                pltpu.async_copy(dst_hbm.at[pl.ds(off, ch)], idx.at[b], isem),
                pltpu.async_copy(et_hbm.at[sid, pl.ds(off, ch)], vals.at[b],
                                 vsem),
            ]

        descs = fire(0)
        for k in range(nch):
            b = k % 2
            for dsc in descs:
                dsc.wait()
            if k + 1 < nch:
                descs = fire(k + 1)

            def body(i, carry):
                iv = idx[b, pl.ds(i * 16, 16)]
                vv = vals[b, pl.ds(i * 16, 16)]
                plsc.addupdate_scatter(acc, [iv], vv)
                return carry

            lax.fori_loop(0, ch // 16, body, 0, unroll=16)

        pltpu.sync_copy(acc, out_hbm.at[cid, sid])

    return scatter_k


_BF = jnp.bfloat16


def _dot(a, b):
    return jnp.dot(a.astype(_BF), b.astype(_BF),
                   preferred_element_type=jnp.float32)


def _edge_body(g_ref, eat_ref, w1c, b1, w2, b2, w3, b3t, lst, lbt, out_ref):
    g = g_ref[...]
    eat = eat_ref[...]
    t1 = lax.dot_general(eat.astype(_BF), w1c[...].astype(_BF),
                         (((0,), (0,)), ((), ())),
                         preferred_element_type=jnp.float32)
    h = g + t1 + b1[...]
    h = jnp.maximum(h, 0.0)
    h = _dot(h, w2[...]) + b2[...]
    h = jnp.maximum(h, 0.0)
    h3t = lax.dot_general(w3[...].astype(_BF), h.astype(_BF),
                          (((0,), (1,)), ((), ())),
                          preferred_element_type=jnp.float32)
    h3t = h3t + b3t[...]
    mu = jnp.mean(h3t, axis=0, keepdims=True)
    hc = h3t - mu
    var = jnp.mean(hc * hc, axis=0, keepdims=True)
    out_ref[...] = eat + hc * lax.rsqrt(var + 1e-5) * lst[...] + lbt[...]


@functools.lru_cache(maxsize=None)
def _make_edge_mlp(n_edges, d, de, h_dim, block_e):
    grid = (n_edges // block_e,)
    full = lambda shape: pl.BlockSpec(shape, lambda i: (0,) * len(shape))
    return pl.pallas_call(
        _edge_body,
        grid=grid,
        in_specs=[
            pl.BlockSpec((block_e, d), lambda i: (i, 0)),
            pl.BlockSpec((de, block_e), lambda i: (0, i)),
            full((de, h_dim)), full((1, h_dim)),
            full((h_dim, h_dim)), full((1, h_dim)),
            full((h_dim, de)), full((de, 1)),
            full((de, 1)), full((de, 1)),
        ],
        out_specs=pl.BlockSpec((de, block_e), lambda i: (0, i)),
        out_shape=jax.ShapeDtypeStruct((de, n_edges), jnp.float32),
    )


def _agg_term(agg_ref, agg2_ref, w1a):
    aggt = agg_ref[0] + agg_ref[1] + agg2_ref[0] + agg2_ref[1]
    return lax.dot_general(aggt.astype(_BF), w1a[...].astype(_BF),
                           (((0,), (0,)), ((), ())),
                           preferred_element_type=jnp.float32)


def _node_body_proj(x_ref, agg_ref, agg2_ref, w1x, w1a, b1, w2, b2, w3, b3,
                    ls, lb, wa, wb, out_ref, pa_ref, pb_ref):
    x = x_ref[...]
    h = _dot(x, w1x[...]) + _agg_term(agg_ref, agg2_ref, w1a) + b1[...]
    h = jnp.maximum(h, 0.0)
    h = _dot(h, w2[...]) + b2[...]
    h = jnp.maximum(h, 0.0)
    h = _dot(h, w3[...]) + b3[...]
    mu = jnp.mean(h, axis=-1, keepdims=True)
    hc = h - mu
    var = jnp.mean(hc * hc, axis=-1, keepdims=True)
    xn = x + hc * lax.rsqrt(var + 1e-5) * ls[...] + lb[...]
    out_ref[...] = xn
    pa_ref[...] = _dot(xn, wa[...])
    pb_ref[...] = _dot(xn, wb[...])


def _node_body_last(x_ref, agg_ref, agg2_ref, w1x, w1a, b1, w2, b2, w3, b3,
                    ls, lb, out_ref):
    x = x_ref[...]
    h = _dot(x, w1x[...]) + _agg_term(agg_ref, agg2_ref, w1a) + b1[...]
    h = jnp.maximum(h, 0.0)
    h = _dot(h, w2[...]) + b2[...]
    h = jnp.maximum(h, 0.0)
    h = _dot(h, w3[...]) + b3[...]
    mu = jnp.mean(h, axis=-1, keepdims=True)
    hc = h - mu
    var = jnp.mean(hc * hc, axis=-1, keepdims=True)
    out_ref[...] = x + hc * lax.rsqrt(var + 1e-5) * ls[...] + lb[...]


@functools.lru_cache(maxsize=None)
def _make_node_mlp(n_nodes, d, de, h_dim, block_n, with_proj):
    grid = (n_nodes // block_n,)
    full = lambda shape: pl.BlockSpec(shape, lambda i: (0,) * len(shape))
    in_specs = [
        pl.BlockSpec((block_n, d), lambda i: (i, 0)),
        full((_NC, de, n_nodes)),
        full((_NC, de, n_nodes)),
        full((d, h_dim)), full((de, h_dim)), full((1, h_dim)),
        full((h_dim, h_dim)), full((1, h_dim)),
        full((h_dim, d)), full((1, d)),
        full((1, d)), full((1, d)),
    ]
    if with_proj:
        in_specs += [full((d, h_dim)), full((d, h_dim))]
        return pl.pallas_call(
            _node_body_proj,
            grid=grid,
            in_specs=in_specs,
            out_specs=[pl.BlockSpec((block_n, d), lambda i: (i, 0))] * 3,
            out_shape=[jax.ShapeDtypeStruct((n_nodes, d), jnp.float32)] * 3,
        )
    return pl.pallas_call(
        _node_body_last,
        grid=grid,
        in_specs=in_specs,
        out_specs=pl.BlockSpec((block_n, d), lambda i: (i, 0)),
        out_shape=jax.ShapeDtypeStruct((n_nodes, d), jnp.float32),
    )


def _proj_body(x_ref, wa, wb, pa_ref, pb_ref):
    x = x_ref[...]
    pa_ref[...] = _dot(x, wa[...])
    pb_ref[...] = _dot(x, wb[...])


@functools.lru_cache(maxsize=None)
def _make_proj(n_nodes, d, h_dim, block_n):
    grid = (n_nodes // block_n,)
    full = lambda shape: pl.BlockSpec(shape, lambda i: (0,) * len(shape))
    return pl.pallas_call(
        _proj_body,
        grid=grid,
        in_specs=[
            pl.BlockSpec((block_n, d), lambda i: (i, 0)),
            full((d, h_dim)), full((d, h_dim)),
        ],
        out_specs=[pl.BlockSpec((block_n, d), lambda i: (i, 0))] * 2,
        out_shape=[jax.ShapeDtypeStruct((n_nodes, d), jnp.float32)] * 2,
    )


def kernel(x, edge_index, edge_attr,
           eW1, eb1, eW2, eb2, eW3, eb3, eLs, eLb,
           nW1, nb1, nW2, nb2, nW3, nb3, nLs, nLb):
    n_nodes, d = x.shape
    n_edges, de = edge_attr.shape
    nb, _, h_dim = eW2.shape
    unit = _GBIG * _NW
    n_a = (n_edges * 16 // 25) // unit * unit
    n_b = n_edges - n_a

    def _gidx(row, lo, n):
        return row[lo:lo + n].reshape(_NW, n // _NW // _GBIG, _GF, _GSUB)

    src_a = _gidx(edge_index[0], 0, n_a)
    dst_a = _gidx(edge_index[1], 0, n_a)
    src_b = _gidx(edge_index[0], n_a, n_b)
    dst_b = _gidx(edge_index[1], n_a, n_b)
    dstf_a = edge_index[1, :n_a]
    dstf_b = edge_index[1, n_a:]
    zeros = jnp.zeros((n_nodes,), jnp.float32)

    eW1a = eW1[:, :d, :]
    eW1b = eW1[:, d:2 * d, :]
    eW1c = eW1[:, 2 * d:, :]
    nW1x = nW1[:, :d, :]
    nW1a = nW1[:, d:, :]
    r1 = lambda a: a.reshape(a.shape[0], 1, a.shape[-1])
    rt = lambda a: a.reshape(a.shape[0], a.shape[-1], 1)
    eb1r, eb2r = map(r1, (eb1, eb2))
    eb3r, eLsr, eLbr = map(rt, (eb3, eLs, eLb))
    nb1r, nb2r, nb3r, nLsr, nLbr = map(r1, (nb1, nb2, nb3, nLs, nLb))

    gather_a = _make_gather(n_a, n_nodes, d)
    gather_b = _make_gather(n_b, n_nodes, d)
    scatter_a = _make_scatter(n_a, n_nodes, de)
    scatter_b = _make_scatter(n_b, n_nodes, de)
    edge_mlp_a = _make_edge_mlp(n_a, d, de, h_dim, 6400)
    edge_mlp_b = _make_edge_mlp(n_b, d, de, h_dim, 6400)
    node_mlp = _make_node_mlp(n_nodes, d, de, h_dim, n_nodes, True)
    node_mlp_last = _make_node_mlp(n_nodes, d, de, h_dim, n_nodes, False)
    proj = _make_proj(n_nodes, d, h_dim, 2000)

    pa, pb = proj(x, eW1a[0], eW1b[0])
    eat = edge_attr.T
    ea_a = eat[:, :n_a]
    ea_b = eat[:, n_a:]
    for i in range(nb):
        ew = (eW1c[i], eb1r[i], eW2[i], eb2r[i], eW3[i], eb3r[i],
              eLsr[i], eLbr[i])
        g_a = gather_a(pa, pb, src_a, dst_a)
        g_b = gather_b(pa, pb, src_b, dst_b)
        ea_a = edge_mlp_a(g_a, ea_a, *ew)
        agg_a = scatter_a(ea_a, dstf_a, zeros)
        ea_b = edge_mlp_b(g_b, ea_b, *ew)
        agg_b = scatter_b(ea_b, dstf_b, zeros)
        if i + 1 < nb:
            x, pa, pb = node_mlp(x, agg_a, agg_b, nW1x[i], nW1a[i], nb1r[i],
                                 nW2[i], nb2r[i], nW3[i], nb3r[i],
                                 nLsr[i], nLbr[i], eW1a[i + 1], eW1b[i + 1])
        else:
            x = node_mlp_last(x, agg_a, agg_b, nW1x[i], nW1a[i], nb1r[i],
                              nW2[i], nb2r[i], nW3[i], nb3r[i],
                              nLsr[i], nLbr[i])
    return x


# speedup vs baseline: 1.0267x; 1.0006x over previous
import functools

import jax
import jax.numpy as jnp
from jax import lax
from jax.experimental import pallas as pl
from jax.experimental.pallas import tpu as pltpu
from jax.experimental.pallas import tpu_sc as plsc

_NC, _NS = 2, 16
_NW = _NC * _NS


_GSUB = 100
_GF = 2
_GBIG = _GSUB * _GF


@functools.lru_cache(maxsize=None)
def _make_gather(n_edges, n_nodes, d):
    epw = n_edges // _NW
    nbig = epw // _GBIG
    mesh = plsc.VectorSubcoreMesh(
        core_axis_name="c", subcore_axis_name="s",
        num_cores=_NC, num_subcores=_NS)

    @functools.partial(
        pl.kernel,
        out_type=jax.ShapeDtypeStruct((n_edges, d), jnp.float32),
        mesh=mesh,
        scratch_types=[
            pltpu.VMEM((2, _GF, _GSUB), jnp.int32),
            pltpu.VMEM((2, _GF, _GSUB), jnp.int32),
            pltpu.VMEM((2, _GBIG, d), jnp.float32),
            pltpu.SemaphoreType.DMA,
            pltpu.SemaphoreType.DMA,
            pltpu.SemaphoreType.DMA,
            pltpu.SemaphoreType.DMA,
        ],
        compiler_params=pltpu.CompilerParams(use_tc_tiling_on_sc=False),
    )
    def gather_k(pa_hbm, pb_hbm, src_hbm, dst_hbm, g_hbm,
                 idx_a, idx_b, rows, isem, asem, bsem, wsem):
        wid = lax.axis_index("s") * _NC + lax.axis_index("c")
        base = pl.multiple_of(wid * epw, 8)

        def fire_idx(k):
            b = k % 2
            return [pltpu.async_copy(src_hbm.at[wid, k], idx_a.at[b], isem),
                    pltpu.async_copy(dst_hbm.at[wid, k], idx_b.at[b], isem)]

        wdesc = [None] * nbig
        idesc = fire_idx(0)
        for k in range(nbig):
            b = k % 2
            if k >= 2:
                wdesc[k - 2].wait()
            for dsc in idesc:
                dsc.wait()
            pa_descs = [
                pltpu.async_copy(
                    pa_hbm.at[idx_a.at[b, j]],
                    rows.at[b, pl.ds(j * _GSUB, _GSUB)], asem)
                for j in range(_GF)
            ]
            if k + 1 < nbig:
                idesc = fire_idx(k + 1)
            pb_descs = []
            for j in range(_GF):
                pa_descs[j].wait()
                pb_descs.append(pltpu.async_copy(
                    pb_hbm.at[idx_b.at[b, j]],
                    rows.at[b, pl.ds(j * _GSUB, _GSUB)], bsem, add=True))
            for dsc in pb_descs:
                dsc.wait()
            off = pl.multiple_of(base + k * _GBIG, 8)
            wdesc[k] = pltpu.async_copy(
                rows.at[b], g_hbm.at[pl.ds(off, _GBIG)], wsem)
        wdesc[nbig - 2].wait()
        wdesc[nbig - 1].wait()

    return gather_k


@functools.lru_cache(maxsize=None)
def _make_scatter(n_edges, n_nodes, de):
    eph = n_edges // _NC
    nch = -(-eph // 8000)
    while eph % nch or (eph // nch) % 16:
        nch += 1
    ch = eph // nch
    mesh = plsc.VectorSubcoreMesh(
        core_axis_name="c", subcore_axis_name="s",
        num_cores=_NC, num_subcores=_NS)

    @functools.partial(
        pl.kernel,
        out_type=jax.ShapeDtypeStruct((_NC, de, n_nodes), jnp.float32),
        mesh=mesh,
        scratch_types=[
            pltpu.VMEM((2, ch), jnp.int32),
            pltpu.VMEM((2, ch), jnp.float32),
            pltpu.VMEM((n_nodes,), jnp.float32),
            pltpu.SemaphoreType.DMA,
            pltpu.SemaphoreType.DMA,
        ],
        compiler_params=pltpu.CompilerParams(use_tc_tiling_on_sc=False,
                                             needs_layout_passes=False),
    )
    def scatter_k(et_hbm, dst_hbm, zeros_hbm, out_hbm, idx, vals, acc,
                  isem, vsem):
        cid = lax.axis_index("c")
        sid = lax.axis_index("s")
        base = cid * eph
        pltpu.sync_copy(zeros_hbm, acc)

        def fire(k):
            b = k % 2
            off = pl.multiple_of(base + k * ch, 8)
            return [
                pltpu.async_copy(dst_hbm.at[pl.ds(off, ch)], idx.at[b], isem),
                pltpu.async_copy(et_hbm.at[sid, pl.ds(off, ch)], vals.at[b],
                                 vsem),
            ]

        descs = fire(0)
        for k in range(nch):
            b = k % 2
            for dsc in descs:
                dsc.wait()
            if k + 1 < nch:
                descs = fire(k + 1)

            def body(i, carry):
                iv = idx[b, pl.ds(i * 16, 16)]
                vv = vals[b, pl.ds(i * 16, 16)]
                plsc.addupdate_scatter(acc, [iv], vv)
                return carry

            lax.fori_loop(0, ch // 16, body, 0, unroll=16)

        pltpu.sync_copy(acc, out_hbm.at[cid, sid])

    return scatter_k


_BF = jnp.bfloat16


def _dot(a, b):
    return jnp.dot(a.astype(_BF), b.astype(_BF),
                   preferred_element_type=jnp.float32)


def _edge_body(g_ref, eat_ref, w1c, b1, w2, b2, w3, b3t, lst, lbt, out_ref):
    g = g_ref[...]
    eat = eat_ref[...]
    t1 = lax.dot_general(eat.astype(_BF), w1c[...].astype(_BF),
                         (((0,), (0,)), ((), ())),
                         preferred_element_type=jnp.float32)
    h = g + t1 + b1[...]
    h = jnp.maximum(h, 0.0)
    h = _dot(h, w2[...]) + b2[...]
    h = jnp.maximum(h, 0.0)
    h3t = lax.dot_general(w3[...].astype(_BF), h.astype(_BF),
                          (((0,), (1,)), ((), ())),
                          preferred_element_type=jnp.float32)
    h3t = h3t + b3t[...]
    mu = jnp.mean(h3t, axis=0, keepdims=True)
    hc = h3t - mu
    var = jnp.mean(hc * hc, axis=0, keepdims=True)
    out_ref[...] = eat + hc * lax.rsqrt(var + 1e-5) * lst[...] + lbt[...]


@functools.lru_cache(maxsize=None)
def _make_edge_mlp(n_edges, d, de, h_dim, block_e):
    grid = (n_edges // block_e,)
    full = lambda shape: pl.BlockSpec(shape, lambda i: (0,) * len(shape))
    return pl.pallas_call(
        _edge_body,
        grid=grid,
        in_specs=[
            pl.BlockSpec((block_e, d), lambda i: (i, 0)),
            pl.BlockSpec((de, block_e), lambda i: (0, i)),
            full((de, h_dim)), full((1, h_dim)),
            full((h_dim, h_dim)), full((1, h_dim)),
            full((h_dim, de)), full((de, 1)),
            full((de, 1)), full((de, 1)),
        ],
        out_specs=pl.BlockSpec((de, block_e), lambda i: (0, i)),
        out_shape=jax.ShapeDtypeStruct((de, n_edges), jnp.float32),
    )


def _agg_term(agg_ref, agg2_ref, w1a):
    aggt = agg_ref[0] + agg_ref[1] + agg2_ref[0] + agg2_ref[1]
    return lax.dot_general(aggt.astype(_BF), w1a[...].astype(_BF),
                           (((0,), (0,)), ((), ())),
                           preferred_element_type=jnp.float32)


def _node_body_proj(x_ref, agg_ref, agg2_ref, w1x, w1a, b1, w2, b2, w3, b3,
                    ls, lb, wa, wb, out_ref, pa_ref, pb_ref):
    x = x_ref[...]
    h = _dot(x, w1x[...]) + _agg_term(agg_ref, agg2_ref, w1a) + b1[...]
    h = jnp.maximum(h, 0.0)
    h = _dot(h, w2[...]) + b2[...]
    h = jnp.maximum(h, 0.0)
    h = _dot(h, w3[...]) + b3[...]
    mu = jnp.mean(h, axis=-1, keepdims=True)
    hc = h - mu
    var = jnp.mean(hc * hc, axis=-1, keepdims=True)
    xn = x + hc * lax.rsqrt(var + 1e-5) * ls[...] + lb[...]
    out_ref[...] = xn
    pa_ref[...] = _dot(xn, wa[...])
    pb_ref[...] = _dot(xn, wb[...])


def _node_body_last(x_ref, agg_ref, agg2_ref, w1x, w1a, b1, w2, b2, w3, b3,
                    ls, lb, out_ref):
    x = x_ref[...]
    h = _dot(x, w1x[...]) + _agg_term(agg_ref, agg2_ref, w1a) + b1[...]
    h = jnp.maximum(h, 0.0)
    h = _dot(h, w2[...]) + b2[...]
    h = jnp.maximum(h, 0.0)
    h = _dot(h, w3[...]) + b3[...]
    mu = jnp.mean(h, axis=-1, keepdims=True)
    hc = h - mu
    var = jnp.mean(hc * hc, axis=-1, keepdims=True)
    out_ref[...] = x + hc * lax.rsqrt(var + 1e-5) * ls[...] + lb[...]


@functools.lru_cache(maxsize=None)
def _make_node_mlp(n_nodes, d, de, h_dim, block_n, with_proj):
    grid = (n_nodes // block_n,)
    full = lambda shape: pl.BlockSpec(shape, lambda i: (0,) * len(shape))
    in_specs = [
        pl.BlockSpec((block_n, d), lambda i: (i, 0)),
        full((_NC, de, n_nodes)),
        full((_NC, de, n_nodes)),
        full((d, h_dim)), full((de, h_dim)), full((1, h_dim)),
        full((h_dim, h_dim)), full((1, h_dim)),
        full((h_dim, d)), full((1, d)),
        full((1, d)), full((1, d)),
    ]
    if with_proj:
        in_specs += [full((d, h_dim)), full((d, h_dim))]
        return pl.pallas_call(
            _node_body_proj,
            grid=grid,
            in_specs=in_specs,
            out_specs=[pl.BlockSpec((block_n, d), lambda i: (i, 0))] * 3,
            out_shape=[jax.ShapeDtypeStruct((n_nodes, d), jnp.float32)] * 3,
        )
    return pl.pallas_call(
        _node_body_last,
        grid=grid,
        in_specs=in_specs,
        out_specs=pl.BlockSpec((block_n, d), lambda i: (i, 0)),
        out_shape=jax.ShapeDtypeStruct((n_nodes, d), jnp.float32),
    )


def _proj_body(x_ref, wa, wb, pa_ref, pb_ref):
    x = x_ref[...]
    pa_ref[...] = _dot(x, wa[...])
    pb_ref[...] = _dot(x, wb[...])


@functools.lru_cache(maxsize=None)
def _make_proj(n_nodes, d, h_dim, block_n):
    grid = (n_nodes // block_n,)
    full = lambda shape: pl.BlockSpec(shape, lambda i: (0,) * len(shape))
    return pl.pallas_call(
        _proj_body,
        grid=grid,
        in_specs=[
            pl.BlockSpec((block_n, d), lambda i: (i, 0)),
            full((d, h_dim)), full((d, h_dim)),
        ],
        out_specs=[pl.BlockSpec((block_n, d), lambda i: (i, 0))] * 2,
        out_shape=[jax.ShapeDtypeStruct((n_nodes, d), jnp.float32)] * 2,
    )


def kernel(x, edge_index, edge_attr,
           eW1, eb1, eW2, eb2, eW3, eb3, eLs, eLb,
           nW1, nb1, nW2, nb2, nW3, nb3, nLs, nLb):
    n_nodes, d = x.shape
    n_edges, de = edge_attr.shape
    nb, _, h_dim = eW2.shape
    unit = _GBIG * _NW
    n_a = (n_edges * 16 // 25) // unit * unit
    n_b = n_edges - n_a

    def _gidx(row, lo, n):
        return row[lo:lo + n].reshape(_NW, n // _NW // _GBIG, _GF, _GSUB)

    src_a = _gidx(edge_index[0], 0, n_a)
    dst_a = _gidx(edge_index[1], 0, n_a)
    src_b = _gidx(edge_index[0], n_a, n_b)
    dst_b = _gidx(edge_index[1], n_a, n_b)
    dstf_a = edge_index[1, :n_a]
    dstf_b = edge_index[1, n_a:]
    zeros = jnp.zeros((n_nodes,), jnp.float32)

    eW1a = eW1[:, :d, :]
    eW1b = eW1[:, d:2 * d, :]
    eW1c = eW1[:, 2 * d:, :]
    nW1x = nW1[:, :d, :]
    nW1a = nW1[:, d:, :]
    r1 = lambda a: a.reshape(a.shape[0], 1, a.shape[-1])
    rt = lambda a: a.reshape(a.shape[0], a.shape[-1], 1)
    eb1r, eb2r = map(r1, (eb1, eb2))
    eb3r, eLsr, eLbr = map(rt, (eb3, eLs, eLb))
    nb1r, nb2r, nb3r, nLsr, nLbr = map(r1, (nb1, nb2, nb3, nLs, nLb))

    gather_a = _make_gather(n_a, n_nodes, d)
    gather_b = _make_gather(n_b, n_nodes, d)
    scatter_a = _make_scatter(n_a, n_nodes, de)
    scatter_b = _make_scatter(n_b, n_nodes, de)
    edge_mlp_a = _make_edge_mlp(n_a, d, de, h_dim, 6400)
    edge_mlp_b = _make_edge_mlp(n_b, d, de, h_dim, 6400)
    node_mlp = _make_node_mlp(n_nodes, d, de, h_dim, n_nodes, True)
    node_mlp_last = _make_node_mlp(n_nodes, d, de, h_dim, n_nodes, False)
    proj = _make_proj(n_nodes, d, h_dim, 2000)

    pa, pb = proj(x, eW1a[0], eW1b[0])
    eat = edge_attr.T
    ea_a = eat[:, :n_a]
    ea_b = eat[:, n_a:]
    for i in range(nb):
        ew = (eW1c[i], eb1r[i], eW2[i], eb2r[i], eW3[i], eb3r[i],
              eLsr[i], eLbr[i])
        g_a = gather_a(pa, pb, src_a, dst_a)
        g_b = gather_b(pa, pb, src_b, dst_b)
        ea_a = edge_mlp_a(g_a, ea_a, *ew)
        agg_a = scatter_a(ea_a, dstf_a, zeros)
        ea_b = edge_mlp_b(g_b, ea_b, *ew)
        agg_b = scatter_b(ea_b, dstf_b, zeros)
        if i + 1 < nb:
            x, pa, pb = node_mlp(x, agg_a, agg_b, nW1x[i], nW1a[i], nb1r[i],
                                 nW2[i], nb2r[i], nW3[i], nb3r[i],
                                 nLsr[i], nLbr[i], eW1a[i + 1], eW1b[i + 1])
        else:
            x = node_mlp_last(x, agg_a, agg_b, nW1x[i], nW1a[i], nb1r[i],
                              nW2[i], nb2r[i], nW3[i], nb3r[i],
                              nLsr[i], nLbr[i])
    return x
